# baseline (speedup 1.0000x reference)
import functools

import jax
import jax.numpy as jnp
from jax import lax
from jax.experimental import pallas as pl
from jax.experimental.pallas import tpu as pltpu
from jax.experimental.pallas import tpu_sc as plsc

_B = 4096
_D = 512
_L = 16
_NW = 32
_ROWS = _B // _NW
_CHUNK = 64


def _sc_body(feat, lab, cent, out, idx_v, c_v, f_v, out_v, gsem):
    wid = lax.axis_index("s") * 2 + lax.axis_index("c")
    base = wid * _ROWS

    pltpu.sync_copy(lab.at[pl.ds(base, _ROWS)], idx_v)
    gather = pltpu.async_copy(cent.at[idx_v], c_v, gsem)
    pltpu.sync_copy(feat.at[pl.ds(base, _CHUNK)], f_v)
    gather.wait()

    acc = jnp.zeros((_L,), jnp.float32)
    for chunk in range(_ROWS // _CHUNK):
        if chunk:
            pltpu.sync_copy(feat.at[pl.ds(base + chunk * _CHUNK, _CHUNK)], f_v)

        def row_body(r, acc, chunk=chunk):
            for d in range(0, _D, _L):
                df = f_v[r, pl.ds(d, _L)] - c_v[chunk * _CHUNK + r, pl.ds(d, _L)]
                acc = acc + df * df
            return acc

        acc = lax.fori_loop(0, _CHUNK, row_body, acc)

    out_v[...] = acc
    pltpu.sync_copy(out_v, out.at[wid])


_sc_call = functools.partial(
    pl.kernel,
    out_type=jax.ShapeDtypeStruct((_NW, _L), jnp.float32),
    mesh=plsc.VectorSubcoreMesh(core_axis_name="c", subcore_axis_name="s"),
    scratch_types=[
        pltpu.VMEM((_ROWS,), jnp.int32),
        pltpu.VMEM((_ROWS, _D), jnp.float32),
        pltpu.VMEM((_CHUNK, _D), jnp.float32),
        pltpu.VMEM((_L,), jnp.float32),
        pltpu.SemaphoreType.DMA,
    ],
)(_sc_body)


@jax.jit
def kernel(features, labels, centers):
    partials = _sc_call(features, labels.astype(jnp.int32), centers)
    return jnp.sum(partials) / (2.0 * features.shape[0])
